# Initial kernel scaffold; baseline (speedup 1.0000x reference)
import jax
import jax.numpy as jnp
from jax.experimental import pallas as pl


def kernel(visn_feats, table):
    raise NotImplementedError("write your pallas kernel here")


# speedup vs baseline: 2.8883x; 2.8883x over previous
import jax
import jax.numpy as jnp
from jax.experimental import pallas as pl


def _body(x_ref, t_ref, o_ref):
    o_ref[...] = x_ref[...] + t_ref[...][None]


def kernel(visn_feats, table):
    B, S, D = visn_feats.shape
    W = table.shape[0]
    SBLK = 512
    t_tiled = jnp.tile(table, (SBLK // W, 1))
    return pl.pallas_call(
        _body,
        grid=(B, S // SBLK),
        in_specs=[
            pl.BlockSpec((1, SBLK, D), lambda b, s: (b, s, 0)),
            pl.BlockSpec((SBLK, D), lambda b, s: (0, 0)),
        ],
        out_specs=pl.BlockSpec((1, SBLK, D), lambda b, s: (b, s, 0)),
        out_shape=jax.ShapeDtypeStruct(visn_feats.shape, visn_feats.dtype),
    )(visn_feats, t_tiled)
